# Initial kernel scaffold; baseline (speedup 1.0000x reference)
import jax
import jax.numpy as jnp
from jax.experimental import pallas as pl


def kernel(context, target, negative, emb):
    raise NotImplementedError("write your pallas kernel here")


# speedup vs baseline: 1.5732x; 1.5732x over previous
import functools

import jax
import jax.numpy as jnp
from jax import lax
from jax.experimental import pallas as pl
from jax.experimental.pallas import tpu as pltpu
from jax.experimental.pallas import tpu_sc as plsc

NC = 2
NS = 16
NW = NC * NS
LANES = 16

B = 16384
D = 56
WIN = 4
NEG = 4
PW = B // NW
CB = 128
NCH = PW // CB
IDX_ROWS = NCH * (WIN + 1 + NEG)


def _sc_body(table, idxs, pos_out, neg_out,
             idx_v, ctx_v, tgt_v, neg_v, pos_t, neg_t, sem):
    w = lax.axis_index("s") * NC + lax.axis_index("c")

    pltpu.sync_copy(idxs.at[w], idx_v)

    lane = lax.iota(jnp.int32, 16)
    ones = jnp.full((16,), 1.0, jnp.float32)
    zeros = jnp.full((16,), 0.0, jnp.float32)
    tailmask = jnp.where(lane >= (48 - 40), ones, zeros)
    scat_base = lane * PW

    def chunk_body(c, _):
        ib = c * 9
        cps = []
        for j in range(WIN):
            cps.append(pltpu.async_copy(
                table.at[idx_v.at[ib + j]],
                ctx_v.at[pl.ds(j * CB, CB)], sem))
        cps.append(pltpu.async_copy(
            table.at[idx_v.at[ib + WIN]], tgt_v, sem))
        for j in range(NEG):
            cps.append(pltpu.async_copy(
                table.at[idx_v.at[ib + WIN + 1 + j]],
                neg_v.at[pl.ds(j * CB, CB)], sem))
        for cp in cps:
            cp.wait()

        def elem_body(e, _):
            col = c * CB + e
            t0 = tgt_v[e, pl.ds(0, 16)]
            t1 = tgt_v[e, pl.ds(16, 16)]
            t2 = tgt_v[e, pl.ds(32, 16)]
            t3 = tgt_v[e, pl.ds(40, 16)]
            r = 4 * e
            c0 = (ctx_v[r, pl.ds(0, 16)] + ctx_v[r + 1, pl.ds(0, 16)]
                  + ctx_v[r + 2, pl.ds(0, 16)] + ctx_v[r + 3, pl.ds(0, 16)])
            c1 = (ctx_v[r, pl.ds(16, 16)] + ctx_v[r + 1, pl.ds(16, 16)]
                  + ctx_v[r + 2, pl.ds(16, 16)] + ctx_v[r + 3, pl.ds(16, 16)])
            c2 = (ctx_v[r, pl.ds(32, 16)] + ctx_v[r + 1, pl.ds(32, 16)]
                  + ctx_v[r + 2, pl.ds(32, 16)] + ctx_v[r + 3, pl.ds(32, 16)])
            c3 = (ctx_v[r, pl.ds(40, 16)] + ctx_v[r + 1, pl.ds(40, 16)]
                  + ctx_v[r + 2, pl.ds(40, 16)] + ctx_v[r + 3, pl.ds(40, 16)])
            pos = c0 * t0 + c1 * t1 + c2 * t2 + (c3 * t3) * tailmask

            n0 = (neg_v[r, pl.ds(0, 16)] + neg_v[r + 1, pl.ds(0, 16)]
                  + neg_v[r + 2, pl.ds(0, 16)] + neg_v[r + 3, pl.ds(0, 16)])
            n1 = (neg_v[r, pl.ds(16, 16)] + neg_v[r + 1, pl.ds(16, 16)]
                  + neg_v[r + 2, pl.ds(16, 16)] + neg_v[r + 3, pl.ds(16, 16)])
            n2 = (neg_v[r, pl.ds(32, 16)] + neg_v[r + 1, pl.ds(32, 16)]
                  + neg_v[r + 2, pl.ds(32, 16)] + neg_v[r + 3, pl.ds(32, 16)])
            n3 = (neg_v[r, pl.ds(40, 16)] + neg_v[r + 1, pl.ds(40, 16)]
                  + neg_v[r + 2, pl.ds(40, 16)] + neg_v[r + 3, pl.ds(40, 16)])
            ng = n0 * t0 + n1 * t1 + n2 * t2 + (n3 * t3) * tailmask

            idx = scat_base + col
            plsc.store_scatter(pos_t, [idx], pos)
            plsc.store_scatter(neg_t, [idx], ng)
            return 0

        lax.fori_loop(0, CB, elem_body, 0)
        return 0

    lax.fori_loop(0, NCH, chunk_body, 0)

    pltpu.sync_copy(pos_t, pos_out.at[w])
    pltpu.sync_copy(neg_t, neg_out.at[w])


_sc_dots = pl.kernel(
    _sc_body,
    out_type=(jax.ShapeDtypeStruct((NW, LANES * PW), jnp.float32),
              jax.ShapeDtypeStruct((NW, LANES * PW), jnp.float32)),
    mesh=plsc.VectorSubcoreMesh(core_axis_name="c", subcore_axis_name="s"),
    compiler_params=pltpu.CompilerParams(
        needs_layout_passes=False, use_tc_tiling_on_sc=False),
    scratch_types=[
        pltpu.VMEM((IDX_ROWS, 128), jnp.int32),
        pltpu.VMEM((WIN * CB, D), jnp.float32),
        pltpu.VMEM((CB, D), jnp.float32),
        pltpu.VMEM((NEG * CB, D), jnp.float32),
        pltpu.VMEM((LANES * PW,), jnp.float32),
        pltpu.VMEM((LANES * PW,), jnp.float32),
        pltpu.SemaphoreType.DMA,
    ],
)


def _tc_body(pos_ref, neg_ref, out_ref):
    p = pos_ref[...].reshape(NW, LANES, PW).sum(axis=1) * (1.0 / WIN)
    n = neg_ref[...].reshape(NW, LANES, PW).sum(axis=1) * (1.0 / NEG)
    pls = jax.nn.log_sigmoid(p)
    nls = jax.nn.log_sigmoid(-n)
    loss = -(jnp.sum(pls) + jnp.sum(nls)) * (1.0 / B)
    out_ref[...] = jnp.reshape(loss, (1, 1))


_tc_reduce = pl.pallas_call(
    _tc_body,
    out_shape=jax.ShapeDtypeStruct((1, 1), jnp.float32),
)


def kernel(context, target, negative, emb):
    ctx = jnp.asarray(context, jnp.int32)
    tgt = jnp.asarray(target, jnp.int32)
    neg = jnp.asarray(negative, jnp.int32)

    ctx_r = ctx.reshape(NW, NCH, WIN, 128)
    tgt_r = tgt.reshape(NW, NCH, 1, 128)
    neg_r = neg.reshape(NW, NCH, NEG, 128)
    allidx = jnp.concatenate([ctx_r, tgt_r, neg_r], axis=2)
    allidx = allidx.reshape(NW, IDX_ROWS, 128)

    embp = jnp.pad(emb, ((0, 0), (0, D - emb.shape[1])))
    pos_p, neg_p = _sc_dots(embp, allidx)
    loss = _tc_reduce(pos_p.reshape(NW * LANES, PW),
                      neg_p.reshape(NW * LANES, PW))
    return loss[0, 0]


# speedup vs baseline: 1.6461x; 1.0463x over previous
import functools

import jax
import jax.numpy as jnp
from jax import lax
from jax.experimental import pallas as pl
from jax.experimental.pallas import tpu as pltpu
from jax.experimental.pallas import tpu_sc as plsc

NC = 2
NS = 16
NW = NC * NS
LANES = 16

B = 16384
D = 56
WIN = 4
NEG = 4
PW = B // NW
CB = 64
NCH = PW // CB


def _sc_body(table, ctx_idx, tgt_idx, neg_idx, pos_out, neg_out,
             ctx_i, tgt_i, neg_i, ctx_v, tgt_v, neg_v, pos_t, neg_t,
             sem_a, sem_b):
    w = lax.axis_index("s") * NC + lax.axis_index("c")

    pltpu.sync_copy(ctx_idx.at[w], ctx_i)
    pltpu.sync_copy(tgt_idx.at[w], tgt_i)
    pltpu.sync_copy(neg_idx.at[w], neg_i)

    lane = lax.iota(jnp.int32, 16)
    ones = jnp.full((16,), 1.0, jnp.float32)
    zeros = jnp.full((16,), 0.0, jnp.float32)
    tailmask = jnp.where(lane >= (48 - 40), ones, zeros)
    scat_base = lane * PW

    sems = (sem_a, sem_b)

    def issue(c):
        buf = c % 2
        sem = sems[buf]
        cv = ctx_v.at[buf]
        nv = neg_v.at[buf]
        return [
            pltpu.async_copy(table.at[ctx_i.at[2 * c]],
                             cv.at[pl.ds(0, 128)], sem),
            pltpu.async_copy(table.at[ctx_i.at[2 * c + 1]],
                             cv.at[pl.ds(128, 128)], sem),
            pltpu.async_copy(table.at[tgt_i.at[c]], tgt_v.at[buf], sem),
            pltpu.async_copy(table.at[neg_i.at[2 * c]],
                             nv.at[pl.ds(0, 128)], sem),
            pltpu.async_copy(table.at[neg_i.at[2 * c + 1]],
                             nv.at[pl.ds(128, 128)], sem),
        ]

    def compute(c):
        buf = c % 2

        def elem_body(e, _):
            col = c * CB + e
            t0 = tgt_v[buf, e, pl.ds(0, 16)]
            t1 = tgt_v[buf, e, pl.ds(16, 16)]
            t2 = tgt_v[buf, e, pl.ds(32, 16)]
            t3 = tgt_v[buf, e, pl.ds(40, 16)]
            r = 4 * e
            c0 = (ctx_v[buf, r, pl.ds(0, 16)] + ctx_v[buf, r + 1, pl.ds(0, 16)]
                  + ctx_v[buf, r + 2, pl.ds(0, 16)]
                  + ctx_v[buf, r + 3, pl.ds(0, 16)])
            c1 = (ctx_v[buf, r, pl.ds(16, 16)]
                  + ctx_v[buf, r + 1, pl.ds(16, 16)]
                  + ctx_v[buf, r + 2, pl.ds(16, 16)]
                  + ctx_v[buf, r + 3, pl.ds(16, 16)])
            c2 = (ctx_v[buf, r, pl.ds(32, 16)]
                  + ctx_v[buf, r + 1, pl.ds(32, 16)]
                  + ctx_v[buf, r + 2, pl.ds(32, 16)]
                  + ctx_v[buf, r + 3, pl.ds(32, 16)])
            c3 = (ctx_v[buf, r, pl.ds(40, 16)]
                  + ctx_v[buf, r + 1, pl.ds(40, 16)]
                  + ctx_v[buf, r + 2, pl.ds(40, 16)]
                  + ctx_v[buf, r + 3, pl.ds(40, 16)])
            pos = c0 * t0 + c1 * t1 + c2 * t2 + (c3 * t3) * tailmask

            n0 = (neg_v[buf, r, pl.ds(0, 16)] + neg_v[buf, r + 1, pl.ds(0, 16)]
                  + neg_v[buf, r + 2, pl.ds(0, 16)]
                  + neg_v[buf, r + 3, pl.ds(0, 16)])
            n1 = (neg_v[buf, r, pl.ds(16, 16)]
                  + neg_v[buf, r + 1, pl.ds(16, 16)]
                  + neg_v[buf, r + 2, pl.ds(16, 16)]
                  + neg_v[buf, r + 3, pl.ds(16, 16)])
            n2 = (neg_v[buf, r, pl.ds(32, 16)]
                  + neg_v[buf, r + 1, pl.ds(32, 16)]
                  + neg_v[buf, r + 2, pl.ds(32, 16)]
                  + neg_v[buf, r + 3, pl.ds(32, 16)])
            n3 = (neg_v[buf, r, pl.ds(40, 16)]
                  + neg_v[buf, r + 1, pl.ds(40, 16)]
                  + neg_v[buf, r + 2, pl.ds(40, 16)]
                  + neg_v[buf, r + 3, pl.ds(40, 16)])
            ng = n0 * t0 + n1 * t1 + n2 * t2 + (n3 * t3) * tailmask

            idx = scat_base + col
            plsc.store_scatter(pos_t, [idx], pos)
            plsc.store_scatter(neg_t, [idx], ng)
            return 0

        lax.fori_loop(0, CB, elem_body, 0)

    cps = issue(0)
    for c in range(NCH):
        nxt = issue(c + 1) if c + 1 < NCH else None
        for cp in cps:
            cp.wait()
        compute(c)
        cps = nxt

    pltpu.sync_copy(pos_t, pos_out.at[w])
    pltpu.sync_copy(neg_t, neg_out.at[w])


_sc_dots = pl.kernel(
    _sc_body,
    out_type=(jax.ShapeDtypeStruct((NW, LANES * PW), jnp.float32),
              jax.ShapeDtypeStruct((NW, LANES * PW), jnp.float32)),
    mesh=plsc.VectorSubcoreMesh(core_axis_name="c", subcore_axis_name="s"),
    compiler_params=pltpu.CompilerParams(
        needs_layout_passes=False, use_tc_tiling_on_sc=False),
    scratch_types=[
        pltpu.VMEM((NCH * WIN * CB // 128, 128), jnp.int32),
        pltpu.VMEM((NCH, CB), jnp.int32),
        pltpu.VMEM((NCH * NEG * CB // 128, 128), jnp.int32),
        pltpu.VMEM((2, WIN * CB, D), jnp.float32),
        pltpu.VMEM((2, CB, D), jnp.float32),
        pltpu.VMEM((2, NEG * CB, D), jnp.float32),
        pltpu.VMEM((LANES * PW,), jnp.float32),
        pltpu.VMEM((LANES * PW,), jnp.float32),
        pltpu.SemaphoreType.DMA,
        pltpu.SemaphoreType.DMA,
    ],
)


def _tc_body(pos_ref, neg_ref, out_ref):
    p = pos_ref[...].reshape(NW, LANES, PW).sum(axis=1) * (1.0 / WIN)
    n = neg_ref[...].reshape(NW, LANES, PW).sum(axis=1) * (1.0 / NEG)
    pls = jax.nn.log_sigmoid(p)
    nls = jax.nn.log_sigmoid(-n)
    loss = -(jnp.sum(pls) + jnp.sum(nls)) * (1.0 / B)
    out_ref[...] = jnp.reshape(loss, (1, 1))


_tc_reduce = pl.pallas_call(
    _tc_body,
    out_shape=jax.ShapeDtypeStruct((1, 1), jnp.float32),
)


def kernel(context, target, negative, emb):
    ctx = jnp.asarray(context, jnp.int32)
    tgt = jnp.asarray(target, jnp.int32)
    neg = jnp.asarray(negative, jnp.int32)

    ctx_r = ctx.reshape(NW, NCH * WIN * CB // 128, 128)
    tgt_r = tgt.reshape(NW, NCH, CB)
    neg_r = neg.reshape(NW, NCH * NEG * CB // 128, 128)

    embp = jnp.pad(emb, ((0, 0), (0, D - emb.shape[1])))
    pos_p, neg_p = _sc_dots(embp, ctx_r, tgt_r, neg_r)
    loss = _tc_reduce(pos_p.reshape(NW * LANES, PW),
                      neg_p.reshape(NW * LANES, PW))
    return loss[0, 0]


# speedup vs baseline: 2.1091x; 1.2813x over previous
import functools

import jax
import jax.numpy as jnp
from jax import lax
from jax.experimental import pallas as pl
from jax.experimental.pallas import tpu as pltpu
from jax.experimental.pallas import tpu_sc as plsc

NC = 2
NS = 16
NW = NC * NS
LANES = 16

B = 16384
D = 128
WIN = 4
NEG = 4
PW = B // NW
CB = 32
NCH = PW // CB


def _sc_body(table, ctx_idx, tgt_idx, neg_idx, pos_out, neg_out,
             ctx_i, tgt_i, neg_i, ctx_v, tgt_v, neg_v, pos_t, neg_t,
             sem_a, sem_b):
    w = lax.axis_index("s") * NC + lax.axis_index("c")

    pltpu.sync_copy(ctx_idx.at[w], ctx_i)
    pltpu.sync_copy(tgt_idx.at[w], tgt_i)
    pltpu.sync_copy(neg_idx.at[w], neg_i)

    lane = lax.iota(jnp.int32, 16)
    scat_base = lane * PW

    sems = (sem_a, sem_b)

    def issue(c):
        buf = c % 2
        sem = sems[buf]
        return [
            pltpu.async_copy(table.at[ctx_i.at[c]], ctx_v.at[buf], sem),
            pltpu.async_copy(table.at[tgt_i.at[c]], tgt_v.at[buf], sem),
            pltpu.async_copy(table.at[neg_i.at[c]], neg_v.at[buf], sem),
        ]

    def compute(c):
        buf = c % 2

        def elem_body(e, _):
            col = c * CB + e
            t0 = tgt_v[buf, e, pl.ds(0, 16)]
            t1 = tgt_v[buf, e, pl.ds(16, 16)]
            t2 = tgt_v[buf, e, pl.ds(32, 16)]
            t3 = tgt_v[buf, e, pl.ds(48, 16)]
            r = 4 * e
            c0 = (ctx_v[buf, r, pl.ds(0, 16)] + ctx_v[buf, r + 1, pl.ds(0, 16)]
                  + ctx_v[buf, r + 2, pl.ds(0, 16)]
                  + ctx_v[buf, r + 3, pl.ds(0, 16)])
            c1 = (ctx_v[buf, r, pl.ds(16, 16)]
                  + ctx_v[buf, r + 1, pl.ds(16, 16)]
                  + ctx_v[buf, r + 2, pl.ds(16, 16)]
                  + ctx_v[buf, r + 3, pl.ds(16, 16)])
            c2 = (ctx_v[buf, r, pl.ds(32, 16)]
                  + ctx_v[buf, r + 1, pl.ds(32, 16)]
                  + ctx_v[buf, r + 2, pl.ds(32, 16)]
                  + ctx_v[buf, r + 3, pl.ds(32, 16)])
            c3 = (ctx_v[buf, r, pl.ds(48, 16)]
                  + ctx_v[buf, r + 1, pl.ds(48, 16)]
                  + ctx_v[buf, r + 2, pl.ds(48, 16)]
                  + ctx_v[buf, r + 3, pl.ds(48, 16)])
            pos = c0 * t0 + c1 * t1 + c2 * t2 + c3 * t3

            n0 = (neg_v[buf, r, pl.ds(0, 16)] + neg_v[buf, r + 1, pl.ds(0, 16)]
                  + neg_v[buf, r + 2, pl.ds(0, 16)]
                  + neg_v[buf, r + 3, pl.ds(0, 16)])
            n1 = (neg_v[buf, r, pl.ds(16, 16)]
                  + neg_v[buf, r + 1, pl.ds(16, 16)]
                  + neg_v[buf, r + 2, pl.ds(16, 16)]
                  + neg_v[buf, r + 3, pl.ds(16, 16)])
            n2 = (neg_v[buf, r, pl.ds(32, 16)]
                  + neg_v[buf, r + 1, pl.ds(32, 16)]
                  + neg_v[buf, r + 2, pl.ds(32, 16)]
                  + neg_v[buf, r + 3, pl.ds(32, 16)])
            n3 = (neg_v[buf, r, pl.ds(48, 16)]
                  + neg_v[buf, r + 1, pl.ds(48, 16)]
                  + neg_v[buf, r + 2, pl.ds(48, 16)]
                  + neg_v[buf, r + 3, pl.ds(48, 16)])
            ng = n0 * t0 + n1 * t1 + n2 * t2 + n3 * t3

            idx = scat_base + col
            plsc.store_scatter(pos_t, [idx], pos)
            plsc.store_scatter(neg_t, [idx], ng)
            return 0

        lax.fori_loop(0, CB, elem_body, 0)

    cps = issue(0)
    for c in range(NCH):
        nxt = issue(c + 1) if c + 1 < NCH else None
        for cp in cps:
            cp.wait()
        compute(c)
        cps = nxt

    pltpu.sync_copy(pos_t, pos_out.at[w])
    pltpu.sync_copy(neg_t, neg_out.at[w])


_sc_dots = pl.kernel(
    _sc_body,
    out_type=(jax.ShapeDtypeStruct((NW, LANES * PW), jnp.float32),
              jax.ShapeDtypeStruct((NW, LANES * PW), jnp.float32)),
    mesh=plsc.VectorSubcoreMesh(core_axis_name="c", subcore_axis_name="s"),
    compiler_params=pltpu.CompilerParams(
        needs_layout_passes=False, use_tc_tiling_on_sc=False),
    scratch_types=[
        pltpu.VMEM((NCH * WIN * CB // 128, 128), jnp.int32),
        pltpu.VMEM((NCH, CB), jnp.int32),
        pltpu.VMEM((NCH * NEG * CB // 128, 128), jnp.int32),
        pltpu.VMEM((2, WIN * CB, D), jnp.float32),
        pltpu.VMEM((2, CB, D), jnp.float32),
        pltpu.VMEM((2, NEG * CB, D), jnp.float32),
        pltpu.VMEM((LANES * PW,), jnp.float32),
        pltpu.VMEM((LANES * PW,), jnp.float32),
        pltpu.SemaphoreType.DMA,
        pltpu.SemaphoreType.DMA,
    ],
)


def _tc_body(pos_ref, neg_ref, out_ref):
    p = pos_ref[...].reshape(NW, LANES, PW).sum(axis=1) * (1.0 / WIN)
    n = neg_ref[...].reshape(NW, LANES, PW).sum(axis=1) * (1.0 / NEG)
    pls = jax.nn.log_sigmoid(p)
    nls = jax.nn.log_sigmoid(-n)
    loss = -(jnp.sum(pls) + jnp.sum(nls)) * (1.0 / B)
    out_ref[...] = jnp.reshape(loss, (1, 1))


_tc_reduce = pl.pallas_call(
    _tc_body,
    out_shape=jax.ShapeDtypeStruct((1, 1), jnp.float32),
)


def kernel(context, target, negative, emb):
    ctx = jnp.asarray(context, jnp.int32)
    tgt = jnp.asarray(target, jnp.int32)
    neg = jnp.asarray(negative, jnp.int32)

    ctx_r = ctx.reshape(NW, NCH * WIN * CB // 128, 128)
    tgt_r = tgt.reshape(NW, NCH, CB)
    neg_r = neg.reshape(NW, NCH * NEG * CB // 128, 128)

    embp = jnp.pad(emb, ((0, 0), (0, D - emb.shape[1])))
    pos_p, neg_p = _sc_dots(embp, ctx_r, tgt_r, neg_r)
    loss = _tc_reduce(pos_p.reshape(NW * LANES, PW),
                      neg_p.reshape(NW * LANES, PW))
    return loss[0, 0]


# speedup vs baseline: 2.1097x; 1.0003x over previous
import functools

import jax
import jax.numpy as jnp
from jax import lax
from jax.experimental import pallas as pl
from jax.experimental.pallas import tpu as pltpu
from jax.experimental.pallas import tpu_sc as plsc

NC = 2
NS = 16
NW = NC * NS
LANES = 16

B = 16384
D = 128
GW = D
WIN = 4
NEG = 4
PW = B // NW
CB = 32
NCH = PW // CB


def _sc_body(table, ctx_idx, tgt_idx, neg_idx, pos_out, neg_out,
             ctx_i, tgt_i, neg_i, ctx_v, tgt_v, neg_v, pos_t, neg_t,
             sem_a, sem_b):
    w = lax.axis_index("s") * NC + lax.axis_index("c")

    pltpu.sync_copy(ctx_idx.at[w], ctx_i)
    pltpu.sync_copy(tgt_idx.at[w], tgt_i)
    pltpu.sync_copy(neg_idx.at[w], neg_i)

    lane = lax.iota(jnp.int32, 16)
    scat_base = lane * PW

    sems = (sem_a, sem_b)

    def issue(c):
        buf = c % 2
        sem = sems[buf]
        return [
            pltpu.async_copy(table.at[ctx_i.at[c]], ctx_v.at[buf], sem),
            pltpu.async_copy(table.at[tgt_i.at[c]], tgt_v.at[buf], sem),
            pltpu.async_copy(table.at[neg_i.at[c]], neg_v.at[buf], sem),
        ]

    def compute(c):
        buf = c % 2

        def elem_body(e, _):
            col = c * CB + e
            t0 = tgt_v[buf, e, pl.ds(0, 16)]
            t1 = tgt_v[buf, e, pl.ds(16, 16)]
            t2 = tgt_v[buf, e, pl.ds(32, 16)]
            t3 = tgt_v[buf, e, pl.ds(48, 16)]
            r = 4 * e
            c0 = (ctx_v[buf, r, pl.ds(0, 16)] + ctx_v[buf, r + 1, pl.ds(0, 16)]
                  + ctx_v[buf, r + 2, pl.ds(0, 16)]
                  + ctx_v[buf, r + 3, pl.ds(0, 16)])
            c1 = (ctx_v[buf, r, pl.ds(16, 16)]
                  + ctx_v[buf, r + 1, pl.ds(16, 16)]
                  + ctx_v[buf, r + 2, pl.ds(16, 16)]
                  + ctx_v[buf, r + 3, pl.ds(16, 16)])
            c2 = (ctx_v[buf, r, pl.ds(32, 16)]
                  + ctx_v[buf, r + 1, pl.ds(32, 16)]
                  + ctx_v[buf, r + 2, pl.ds(32, 16)]
                  + ctx_v[buf, r + 3, pl.ds(32, 16)])
            c3 = (ctx_v[buf, r, pl.ds(48, 16)]
                  + ctx_v[buf, r + 1, pl.ds(48, 16)]
                  + ctx_v[buf, r + 2, pl.ds(48, 16)]
                  + ctx_v[buf, r + 3, pl.ds(48, 16)])
            pos = c0 * t0 + c1 * t1 + c2 * t2 + c3 * t3

            n0 = (neg_v[buf, r, pl.ds(0, 16)] + neg_v[buf, r + 1, pl.ds(0, 16)]
                  + neg_v[buf, r + 2, pl.ds(0, 16)]
                  + neg_v[buf, r + 3, pl.ds(0, 16)])
            n1 = (neg_v[buf, r, pl.ds(16, 16)]
                  + neg_v[buf, r + 1, pl.ds(16, 16)]
                  + neg_v[buf, r + 2, pl.ds(16, 16)]
                  + neg_v[buf, r + 3, pl.ds(16, 16)])
            n2 = (neg_v[buf, r, pl.ds(32, 16)]
                  + neg_v[buf, r + 1, pl.ds(32, 16)]
                  + neg_v[buf, r + 2, pl.ds(32, 16)]
                  + neg_v[buf, r + 3, pl.ds(32, 16)])
            n3 = (neg_v[buf, r, pl.ds(48, 16)]
                  + neg_v[buf, r + 1, pl.ds(48, 16)]
                  + neg_v[buf, r + 2, pl.ds(48, 16)]
                  + neg_v[buf, r + 3, pl.ds(48, 16)])
            ng = n0 * t0 + n1 * t1 + n2 * t2 + n3 * t3

            idx = scat_base + col
            plsc.store_scatter(pos_t, [idx], pos)
            plsc.store_scatter(neg_t, [idx], ng)
            return 0

        lax.fori_loop(0, CB, elem_body, 0)

    cps = issue(0)
    for c in range(NCH):
        nxt = issue(c + 1) if c + 1 < NCH else None
        for cp in cps:
            cp.wait()
        compute(c)
        cps = nxt

    pltpu.sync_copy(pos_t, pos_out.at[w])
    pltpu.sync_copy(neg_t, neg_out.at[w])


_sc_dots = pl.kernel(
    _sc_body,
    out_type=(jax.ShapeDtypeStruct((NW, LANES * PW), jnp.float32),
              jax.ShapeDtypeStruct((NW, LANES * PW), jnp.float32)),
    mesh=plsc.VectorSubcoreMesh(core_axis_name="c", subcore_axis_name="s"),
    compiler_params=pltpu.CompilerParams(
        needs_layout_passes=False, use_tc_tiling_on_sc=False),
    scratch_types=[
        pltpu.VMEM((NCH * WIN * CB // 128, 128), jnp.int32),
        pltpu.VMEM((NCH, CB), jnp.int32),
        pltpu.VMEM((NCH * NEG * CB // 128, 128), jnp.int32),
        pltpu.VMEM((2, WIN * CB, GW), jnp.float32),
        pltpu.VMEM((2, CB, GW), jnp.float32),
        pltpu.VMEM((2, NEG * CB, GW), jnp.float32),
        pltpu.VMEM((LANES * PW,), jnp.float32),
        pltpu.VMEM((LANES * PW,), jnp.float32),
        pltpu.SemaphoreType.DMA,
        pltpu.SemaphoreType.DMA,
    ],
)


def _tc_body(pos_ref, neg_ref, out_ref):
    p = pos_ref[...].reshape(NW, LANES, PW).sum(axis=1) * (1.0 / WIN)
    n = neg_ref[...].reshape(NW, LANES, PW).sum(axis=1) * (1.0 / NEG)
    pls = jax.nn.log_sigmoid(p)
    nls = jax.nn.log_sigmoid(-n)
    loss = -(jnp.sum(pls) + jnp.sum(nls)) * (1.0 / B)
    out_ref[...] = jnp.reshape(loss, (1, 1))


_tc_reduce = pl.pallas_call(
    _tc_body,
    out_shape=jax.ShapeDtypeStruct((1, 1), jnp.float32),
)


def kernel(context, target, negative, emb):
    ctx = jnp.asarray(context, jnp.int32)
    tgt = jnp.asarray(target, jnp.int32)
    neg = jnp.asarray(negative, jnp.int32)

    ctx_r = ctx.reshape(NW, NCH * WIN * CB // 128, 128)
    tgt_r = tgt.reshape(NW, NCH, CB)
    neg_r = neg.reshape(NW, NCH * NEG * CB // 128, 128)

    embp = jnp.pad(emb, ((0, 0), (0, D - emb.shape[1])))
    pos_p, neg_p = _sc_dots(embp, ctx_r, tgt_r, neg_r)
    loss = _tc_reduce(pos_p.reshape(NW * LANES, PW),
                      neg_p.reshape(NW * LANES, PW))
    return loss[0, 0]


# speedup vs baseline: 2.1163x; 1.0031x over previous
import functools

import jax
import jax.numpy as jnp
from jax import lax
from jax.experimental import pallas as pl
from jax.experimental.pallas import tpu as pltpu
from jax.experimental.pallas import tpu_sc as plsc

NC = 2
NS = 16
NW = NC * NS
LANES = 16

B = 16384
D = 128
GW = D
WIN = 4
NEG = 4
PW = B // NW
CB = 32
NCH = PW // CB


def _sc_body(table, ctx_idx, tgt_idx, neg_idx, pos_out, neg_out,
             ctx_i, tgt_i, neg_i, ctx_v, tgt_v, neg_v, pos_t, neg_t,
             sem_a, sem_b):
    w = lax.axis_index("s") * NC + lax.axis_index("c")

    pltpu.sync_copy(ctx_idx.at[w], ctx_i)
    pltpu.sync_copy(tgt_idx.at[w], tgt_i)
    pltpu.sync_copy(neg_idx.at[w], neg_i)

    lane = lax.iota(jnp.int32, 16)
    scat_base = lane * PW

    sems = (sem_a, sem_b)

    def issue(c):
        buf = c % 2
        sem = sems[buf]
        return [
            pltpu.async_copy(table.at[ctx_i.at[c]], ctx_v.at[buf], sem),
            pltpu.async_copy(table.at[tgt_i.at[pl.ds(c * CB, CB)]],
                             tgt_v.at[buf], sem),
            pltpu.async_copy(table.at[neg_i.at[c]], neg_v.at[buf], sem),
        ]

    def compute(c):
        buf = c % 2

        def elem_body(e, _):
            col = c * CB + e
            t0 = tgt_v[buf, e, pl.ds(0, 16)]
            t1 = tgt_v[buf, e, pl.ds(16, 16)]
            t2 = tgt_v[buf, e, pl.ds(32, 16)]
            t3 = tgt_v[buf, e, pl.ds(48, 16)]
            r = 4 * e
            c0 = (ctx_v[buf, r, pl.ds(0, 16)] + ctx_v[buf, r + 1, pl.ds(0, 16)]
                  + ctx_v[buf, r + 2, pl.ds(0, 16)]
                  + ctx_v[buf, r + 3, pl.ds(0, 16)])
            c1 = (ctx_v[buf, r, pl.ds(16, 16)]
                  + ctx_v[buf, r + 1, pl.ds(16, 16)]
                  + ctx_v[buf, r + 2, pl.ds(16, 16)]
                  + ctx_v[buf, r + 3, pl.ds(16, 16)])
            c2 = (ctx_v[buf, r, pl.ds(32, 16)]
                  + ctx_v[buf, r + 1, pl.ds(32, 16)]
                  + ctx_v[buf, r + 2, pl.ds(32, 16)]
                  + ctx_v[buf, r + 3, pl.ds(32, 16)])
            c3 = (ctx_v[buf, r, pl.ds(48, 16)]
                  + ctx_v[buf, r + 1, pl.ds(48, 16)]
                  + ctx_v[buf, r + 2, pl.ds(48, 16)]
                  + ctx_v[buf, r + 3, pl.ds(48, 16)])
            pos = c0 * t0 + c1 * t1 + c2 * t2 + c3 * t3

            n0 = (neg_v[buf, r, pl.ds(0, 16)] + neg_v[buf, r + 1, pl.ds(0, 16)]
                  + neg_v[buf, r + 2, pl.ds(0, 16)]
                  + neg_v[buf, r + 3, pl.ds(0, 16)])
            n1 = (neg_v[buf, r, pl.ds(16, 16)]
                  + neg_v[buf, r + 1, pl.ds(16, 16)]
                  + neg_v[buf, r + 2, pl.ds(16, 16)]
                  + neg_v[buf, r + 3, pl.ds(16, 16)])
            n2 = (neg_v[buf, r, pl.ds(32, 16)]
                  + neg_v[buf, r + 1, pl.ds(32, 16)]
                  + neg_v[buf, r + 2, pl.ds(32, 16)]
                  + neg_v[buf, r + 3, pl.ds(32, 16)])
            n3 = (neg_v[buf, r, pl.ds(48, 16)]
                  + neg_v[buf, r + 1, pl.ds(48, 16)]
                  + neg_v[buf, r + 2, pl.ds(48, 16)]
                  + neg_v[buf, r + 3, pl.ds(48, 16)])
            ng = n0 * t0 + n1 * t1 + n2 * t2 + n3 * t3

            idx = scat_base + col
            plsc.store_scatter(pos_t, [idx], pos)
            plsc.store_scatter(neg_t, [idx], ng)
            return 0

        lax.fori_loop(0, CB, elem_body, 0)

    cps = issue(0)
    for c in range(NCH):
        nxt = issue(c + 1) if c + 1 < NCH else None
        for cp in cps:
            cp.wait()
        compute(c)
        cps = nxt

    pltpu.sync_copy(pos_t, pos_out.at[w])
    pltpu.sync_copy(neg_t, neg_out.at[w])


_sc_dots = pl.kernel(
    _sc_body,
    out_type=(jax.ShapeDtypeStruct((NW, LANES * PW), jnp.float32),
              jax.ShapeDtypeStruct((NW, LANES * PW), jnp.float32)),
    mesh=plsc.VectorSubcoreMesh(core_axis_name="c", subcore_axis_name="s"),
    compiler_params=pltpu.CompilerParams(
        needs_layout_passes=False, use_tc_tiling_on_sc=False),
    scratch_types=[
        pltpu.VMEM((NCH * WIN * CB // 128, 128), jnp.int32),
        pltpu.VMEM((PW,), jnp.int32),
        pltpu.VMEM((NCH * NEG * CB // 128, 128), jnp.int32),
        pltpu.VMEM((2, WIN * CB, GW), jnp.float32),
        pltpu.VMEM((2, CB, GW), jnp.float32),
        pltpu.VMEM((2, NEG * CB, GW), jnp.float32),
        pltpu.VMEM((LANES * PW,), jnp.float32),
        pltpu.VMEM((LANES * PW,), jnp.float32),
        pltpu.SemaphoreType.DMA,
        pltpu.SemaphoreType.DMA,
    ],
)


def _tc_body(pos_ref, neg_ref, out_ref):
    p = pos_ref[...].reshape(NW, LANES, PW).sum(axis=1) * (1.0 / WIN)
    n = neg_ref[...].reshape(NW, LANES, PW).sum(axis=1) * (1.0 / NEG)
    pls = jax.nn.log_sigmoid(p)
    nls = jax.nn.log_sigmoid(-n)
    loss = -(jnp.sum(pls) + jnp.sum(nls)) * (1.0 / B)
    out_ref[...] = jnp.reshape(loss, (1, 1))


_tc_reduce = pl.pallas_call(
    _tc_body,
    out_shape=jax.ShapeDtypeStruct((1, 1), jnp.float32),
)


def kernel(context, target, negative, emb):
    ctx = jnp.asarray(context, jnp.int32)
    tgt = jnp.asarray(target, jnp.int32)
    neg = jnp.asarray(negative, jnp.int32)

    ctx_r = ctx.reshape(NW, NCH * WIN * CB // 128, 128)
    tgt_r = tgt.reshape(NW, PW)
    neg_r = neg.reshape(NW, NCH * NEG * CB // 128, 128)

    embp = jnp.pad(emb, ((0, 0), (0, D - emb.shape[1])))
    pos_p, neg_p = _sc_dots(embp, ctx_r, tgt_r, neg_r)
    loss = _tc_reduce(pos_p.reshape(NW * LANES, PW),
                      neg_p.reshape(NW * LANES, PW))
    return loss[0, 0]
